# Initial kernel scaffold; baseline (speedup 1.0000x reference)
import jax
import jax.numpy as jnp
from jax.experimental import pallas as pl


def kernel(z, edge_index, W, b):
    raise NotImplementedError("write your pallas kernel here")


# speedup vs baseline: 5.4938x; 5.4938x over previous
import functools

import jax
import jax.numpy as jnp
from jax import lax
from jax.experimental import pallas as pl
from jax.experimental.pallas import tpu as pltpu
from jax.experimental.pallas import tpu_sc as plsc

_LANES = 16
_LN2 = 0.6931471805599453
_SQRT2 = 1.4142135381698608


def _proj_body(w_ref, z_ref, bp_ref, out_ref):
    out_ref[...] = lax.dot_general(
        w_ref[...], z_ref[...],
        dimension_numbers=(((1,), (1,)), ((), ())),
        preferred_element_type=jnp.float32,
    ) + bp_ref[...]


def _make_sc_gather(n_nodes, n_edges):
    info = plsc.get_sparse_core_info()
    nc, ns = info.num_cores, info.num_subcores
    nw = nc * ns
    assert n_edges % (nw * _LANES) == 0
    e_per_w = n_edges // nw
    n_iter = e_per_w // _LANES
    mesh = plsc.VectorSubcoreMesh(core_axis_name="c", subcore_axis_name="s")

    @functools.partial(
        pl.kernel,
        out_type=jax.ShapeDtypeStruct((n_edges * 3,), jnp.float32),
        mesh=mesh,
        scratch_types=[
            pltpu.VMEM((6 * n_nodes,), jnp.float32),
            pltpu.VMEM((e_per_w,), jnp.int32),
            pltpu.VMEM((e_per_w,), jnp.int32),
            pltpu.VMEM((3 * e_per_w,), jnp.float32),
        ],
        compiler_params=pltpu.CompilerParams(needs_layout_passes=False),
    )
    def sc_gather(tab_hbm, e0_hbm, e1_hbm, out_hbm, tab_v, e0_v, e1_v, out_v):
        wid = lax.axis_index("s") * nc + lax.axis_index("c")
        base = wid * e_per_w
        pltpu.sync_copy(tab_hbm.at[pl.ds(0, 6 * n_nodes)], tab_v)
        pltpu.sync_copy(e0_hbm.at[pl.ds(base, e_per_w)], e0_v)
        pltpu.sync_copy(e1_hbm.at[pl.ds(base, e_per_w)], e1_v)

        iota3 = lax.iota(jnp.int32, _LANES) * 3

        def body(i, carry):
            off = i * _LANES
            ev0 = e0_v[pl.ds(off, _LANES)]
            ev1 = e1_v[pl.ds(off, _LANES)]
            s0 = plsc.load_gather(tab_v, [ev0])
            s1 = plsc.load_gather(tab_v, [ev0 + n_nodes])
            s2 = plsc.load_gather(tab_v, [ev0 + 2 * n_nodes])
            d0 = plsc.load_gather(tab_v, [ev1 + 3 * n_nodes])
            d1 = plsc.load_gather(tab_v, [ev1 + 4 * n_nodes])
            d2 = plsc.load_gather(tab_v, [ev1 + 5 * n_nodes])
            l0, l1, l2 = s0 + d0, s1 + d1, s2 + d2
            m = jnp.maximum(jnp.maximum(l0, l1), l2)
            ssum = (jnp.exp(l0 - m) + jnp.exp(l1 - m) + jnp.exp(l2 - m))
            bits = lax.bitcast_convert_type(ssum, jnp.int32)
            ex = (bits >> 23) - 127
            mf = lax.bitcast_convert_type(
                (bits & 0x007FFFFF) | 0x3F800000, jnp.float32)
            big = mf > _SQRT2
            mf = jnp.where(big, mf * 0.5, mf)
            exf = (ex + big.astype(jnp.int32)).astype(jnp.float32)
            t = (mf - 1.0) / (mf + 1.0)
            t2 = t * t
            p = ((t2 * (1.0 / 7.0) + (1.0 / 5.0)) * t2 + (1.0 / 3.0)) * t2 + 1.0
            lse = m + exf * _LN2 + 2.0 * t * p
            pos = iota3 + off * 3
            plsc.store_scatter(out_v, [pos], l0 - lse)
            plsc.store_scatter(out_v, [pos + 1], l1 - lse)
            plsc.store_scatter(out_v, [pos + 2], l2 - lse)
            return carry

        lax.fori_loop(0, n_iter, body, 0)
        pltpu.sync_copy(out_v, out_hbm.at[pl.ds(base * 3, 3 * e_per_w)])

    return sc_gather


def kernel(z, edge_index, W, b):
    n_nodes, hidden = z.shape
    n_edges = edge_index.shape[1]
    wcat = jnp.concatenate(
        [W[:, :hidden], W[:, hidden:], jnp.zeros((2, hidden), jnp.float32)], 0)
    bp = jnp.concatenate([b, jnp.zeros((5,), jnp.float32)]).reshape(8, 1)
    table = pl.pallas_call(
        _proj_body,
        out_shape=jax.ShapeDtypeStruct((8, n_nodes), jnp.float32),
    )(wcat, z, bp)

    e = edge_index.astype(jnp.int32)
    out_flat = _make_sc_gather(n_nodes, n_edges)(
        table.reshape(-1), e[0], e[1])
    return out_flat.reshape(n_edges, 3)


# speedup vs baseline: 20.2090x; 3.6785x over previous
import functools

import jax
import jax.numpy as jnp
from jax import lax
from jax.experimental import pallas as pl
from jax.experimental.pallas import tpu as pltpu
from jax.experimental.pallas import tpu_sc as plsc

_LANES = 16
_LN2 = 0.6931471805599453
_SQRT2 = 1.4142135381698608


def _proj_body(w_ref, z_ref, bp_ref, out_ref):
    out_ref[...] = lax.dot_general(
        w_ref[...], z_ref[...],
        dimension_numbers=(((1,), (1,)), ((), ())),
        preferred_element_type=jnp.float32,
    ) + bp_ref[...]


def _make_sc_gather(n_nodes, n_edges):
    info = plsc.get_sparse_core_info()
    nc, ns = info.num_cores, info.num_subcores
    nw = nc * ns
    blk = 128
    n_blocks = n_edges // blk
    base_blocks = n_blocks // nw
    n_extra = n_blocks - base_blocks * nw
    e_per_w = base_blocks * blk
    n_iter = e_per_w // _LANES
    mesh = plsc.VectorSubcoreMesh(core_axis_name="c", subcore_axis_name="s")

    @functools.partial(
        pl.kernel,
        out_type=jax.ShapeDtypeStruct((4, n_edges), jnp.float32),
        mesh=mesh,
        scratch_types=[
            pltpu.VMEM((6 * n_nodes,), jnp.float32),
            pltpu.VMEM((e_per_w,), jnp.int32),
            pltpu.VMEM((e_per_w,), jnp.int32),
            pltpu.VMEM((4, e_per_w), jnp.float32),
            pltpu.VMEM((blk,), jnp.int32),
            pltpu.VMEM((blk,), jnp.int32),
            pltpu.VMEM((4, blk), jnp.float32),
        ],
        compiler_params=pltpu.CompilerParams(needs_layout_passes=False),
    )
    def sc_gather(tab_hbm, e0_hbm, e1_hbm, out_hbm,
                  tab_v, e0_v, e1_v, out_v, e0t_v, e1t_v, outt_v):
        wid = lax.axis_index("s") * nc + lax.axis_index("c")
        base = wid * e_per_w
        pltpu.sync_copy(tab_hbm.at[pl.ds(0, 6 * n_nodes)], tab_v)
        pltpu.sync_copy(e0_hbm.at[pl.ds(base, e_per_w)], e0_v)
        pltpu.sync_copy(e1_hbm.at[pl.ds(base, e_per_w)], e1_v)

        def compute(ev0, ev1, o_ref, off):
            s0 = plsc.load_gather(tab_v, [ev0])
            s1 = plsc.load_gather(tab_v, [ev0 + n_nodes])
            s2 = plsc.load_gather(tab_v, [ev0 + 2 * n_nodes])
            d0 = plsc.load_gather(tab_v, [ev1 + 3 * n_nodes])
            d1 = plsc.load_gather(tab_v, [ev1 + 4 * n_nodes])
            d2 = plsc.load_gather(tab_v, [ev1 + 5 * n_nodes])
            l0, l1, l2 = s0 + d0, s1 + d1, s2 + d2
            m = jnp.maximum(jnp.maximum(l0, l1), l2)
            ssum = (jnp.exp(l0 - m) + jnp.exp(l1 - m) + jnp.exp(l2 - m))
            bits = lax.bitcast_convert_type(ssum, jnp.int32)
            ex = (bits >> 23) - 127
            mf = lax.bitcast_convert_type(
                (bits & 0x007FFFFF) | 0x3F800000, jnp.float32)
            big = mf > _SQRT2
            mf = jnp.where(big, mf * 0.5, mf)
            exf = (ex + big.astype(jnp.int32)).astype(jnp.float32)
            t = (mf - 1.0) / (mf + 1.0)
            t2 = t * t
            p = ((t2 * (1.0 / 7.0) + (1.0 / 5.0)) * t2 + (1.0 / 3.0)) * t2 + 1.0
            lse = m + exf * _LN2 + 2.0 * t * p
            sl = pl.ds(off, _LANES)
            o_ref[0, sl] = l0 - lse
            o_ref[1, sl] = l1 - lse
            o_ref[2, sl] = l2 - lse

        def body(i, carry):
            off = i * _LANES
            compute(e0_v[pl.ds(off, _LANES)], e1_v[pl.ds(off, _LANES)],
                    out_v, off)
            return carry

        lax.fori_loop(0, n_iter, body, 0)
        pltpu.sync_copy(out_v, out_hbm.at[:, pl.ds(base, e_per_w)])

        @pl.when(wid >= nw - n_extra)
        def _tail():
            tbase = base_blocks * nw * blk + (wid - (nw - n_extra)) * blk
            pltpu.sync_copy(e0_hbm.at[pl.ds(tbase, blk)], e0t_v)
            pltpu.sync_copy(e1_hbm.at[pl.ds(tbase, blk)], e1t_v)

            def tbody(i, carry):
                off = i * _LANES
                compute(e0t_v[pl.ds(off, _LANES)], e1t_v[pl.ds(off, _LANES)],
                        outt_v, off)
                return carry

            lax.fori_loop(0, blk // _LANES, tbody, 0)
            pltpu.sync_copy(outt_v, out_hbm.at[:, pl.ds(tbase, blk)])

    return sc_gather


def kernel(z, edge_index, W, b):
    n_nodes, hidden = z.shape
    n_edges = edge_index.shape[1]
    wcat = jnp.concatenate(
        [W[:, :hidden], W[:, hidden:], jnp.zeros((2, hidden), jnp.float32)], 0)
    bp = jnp.concatenate([b, jnp.zeros((5,), jnp.float32)]).reshape(8, 1)
    table = pl.pallas_call(
        _proj_body,
        out_shape=jax.ShapeDtypeStruct((8, n_nodes), jnp.float32),
    )(wcat, z, bp)

    e = edge_index.astype(jnp.int32)
    planes = _make_sc_gather(n_nodes, n_edges)(
        table.reshape(-1), e[0], e[1])
    return planes[:3].T


# speedup vs baseline: 25.1828x; 1.2461x over previous
import functools

import jax
import jax.numpy as jnp
from jax import lax
from jax.experimental import pallas as pl
from jax.experimental.pallas import tpu as pltpu
from jax.experimental.pallas import tpu_sc as plsc

_LANES = 16
_BLK = 128
_LN2 = 0.6931471805599453
_SQRT2 = 1.4142135381698608


def _proj_body(w_ref, z_ref, b_ref, out_ref):
    w = w_ref[...]
    z = z_ref[...]
    h = w.shape[1] // 2
    dn = (((1,), (1,)), ((), ()))
    s = lax.dot_general(w[:, :h], z, dn, preferred_element_type=jnp.float32)
    d = lax.dot_general(w[:, h:], z, dn, preferred_element_type=jnp.float32)
    sb = s + b_ref[...][:, None]
    pad = jnp.zeros((2, s.shape[1]), jnp.float32)
    out_ref[...] = jnp.concatenate([sb, d, pad], axis=0)


def _make_sc_gather(n_nodes, n_edges):
    info = plsc.get_sparse_core_info()
    nc, ns = info.num_cores, info.num_subcores
    nw = nc * ns
    n_blocks = n_edges // _BLK
    base_blocks = n_blocks // nw
    n_extra = n_blocks - base_blocks * nw
    e_per_w = base_blocks * _BLK
    mesh = plsc.VectorSubcoreMesh(core_axis_name="c", subcore_axis_name="s")

    @functools.partial(
        pl.kernel,
        out_type=jax.ShapeDtypeStruct((4, n_edges), jnp.float32),
        mesh=mesh,
        scratch_types=[
            pltpu.VMEM((6 * n_nodes,), jnp.float32),
            pltpu.VMEM((base_blocks, 2, _BLK), jnp.int32),
            pltpu.VMEM((4, e_per_w), jnp.float32),
            pltpu.VMEM((1, 2, _BLK), jnp.int32),
            pltpu.VMEM((4, _BLK), jnp.float32),
            pltpu.SemaphoreType.DMA,
            pltpu.SemaphoreType.DMA,
        ],
        compiler_params=pltpu.CompilerParams(needs_layout_passes=False),
    )
    def sc_gather(tab_hbm, ei_hbm, out_hbm,
                  tab_v, ei_v, out_v, eit_v, outt_v, sem0, sem1):
        wid = lax.axis_index("s") * nc + lax.axis_index("c")
        bbase = wid * base_blocks
        cp_tab = pltpu.async_copy(tab_hbm.at[pl.ds(0, 6 * n_nodes)], tab_v,
                                  sem0)
        cp_idx = pltpu.async_copy(ei_hbm.at[pl.ds(bbase, base_blocks)], ei_v,
                                  sem1)
        cp_tab.wait()
        cp_idx.wait()

        def compute16(ev0, ev1, o_ref, col):
            s0 = plsc.load_gather(tab_v, [ev0])
            s1 = plsc.load_gather(tab_v, [ev0 + n_nodes])
            s2 = plsc.load_gather(tab_v, [ev0 + 2 * n_nodes])
            d0 = plsc.load_gather(tab_v, [ev1 + 3 * n_nodes])
            d1 = plsc.load_gather(tab_v, [ev1 + 4 * n_nodes])
            d2 = plsc.load_gather(tab_v, [ev1 + 5 * n_nodes])
            l0, l1, l2 = s0 + d0, s1 + d1, s2 + d2
            m = jnp.maximum(jnp.maximum(l0, l1), l2)
            ssum = (jnp.exp(l0 - m) + jnp.exp(l1 - m) + jnp.exp(l2 - m))
            bits = lax.bitcast_convert_type(ssum, jnp.int32)
            ex = (bits >> 23) - 127
            mf = lax.bitcast_convert_type(
                (bits & 0x007FFFFF) | 0x3F800000, jnp.float32)
            big = mf > _SQRT2
            mf = jnp.where(big, mf * 0.5, mf)
            exf = (ex + big.astype(jnp.int32)).astype(jnp.float32)
            t = (mf - 1.0) / (mf + 1.0)
            t2 = t * t
            p = ((t2 * (1.0 / 7.0) + (1.0 / 5.0)) * t2 + (1.0 / 3.0)) * t2 + 1.0
            lse = m + exf * _LN2 + 2.0 * t * p
            sl = pl.ds(col, _LANES)
            o_ref[0, sl] = l0 - lse
            o_ref[1, sl] = l1 - lse
            o_ref[2, sl] = l2 - lse

        def blk_body(b, carry):
            for g in range(_BLK // _LANES):
                ev0 = ei_v[b, 0, pl.ds(g * _LANES, _LANES)]
                ev1 = ei_v[b, 1, pl.ds(g * _LANES, _LANES)]
                compute16(ev0, ev1, out_v, b * _BLK + g * _LANES)
            return carry

        lax.fori_loop(0, base_blocks, blk_body, 0)
        cp_out = pltpu.async_copy(
            out_v, out_hbm.at[:, pl.ds(bbase * _BLK, e_per_w)], sem1)

        @pl.when(wid >= nw - n_extra)
        def _tail():
            tb = base_blocks * nw + (wid - (nw - n_extra))
            pltpu.async_copy(ei_hbm.at[pl.ds(tb, 1)], eit_v, sem0).wait()
            for g in range(_BLK // _LANES):
                ev0 = eit_v[0, 0, pl.ds(g * _LANES, _LANES)]
                ev1 = eit_v[0, 1, pl.ds(g * _LANES, _LANES)]
                compute16(ev0, ev1, outt_v, g * _LANES)
            pltpu.async_copy(
                outt_v, out_hbm.at[:, pl.ds(tb * _BLK, _BLK)], sem0).wait()

        cp_out.wait()

    return sc_gather


def kernel(z, edge_index, W, b):
    n_nodes, hidden = z.shape
    n_edges = edge_index.shape[1]
    table = pl.pallas_call(
        _proj_body,
        out_shape=jax.ShapeDtypeStruct((8, n_nodes), jnp.float32),
    )(W, z, b)

    ei3 = (edge_index.astype(jnp.int32)
           .reshape(2, n_edges // _BLK, _BLK).transpose(1, 0, 2))
    planes = _make_sc_gather(n_nodes, n_edges)(table.reshape(-1), ei3)
    return planes[:3].T


# speedup vs baseline: 36.7484x; 1.4593x over previous
import functools

import jax
import jax.numpy as jnp
from jax import lax
from jax.experimental import pallas as pl
from jax.experimental.pallas import tpu as pltpu
from jax.experimental.pallas import tpu_sc as plsc

_LANES = 16
_BLK = 128
_LN2 = 0.6931471805599453
_SQRT2 = 1.4142135381698608


def _proj_body(w_ref, z_ref, b_ref, out_ref):
    w = w_ref[...]
    z = z_ref[...]
    h = w.shape[1] // 2
    dn = (((1,), (1,)), ((), ()))
    s = lax.dot_general(w[:, :h], z, dn, preferred_element_type=jnp.float32)
    d = lax.dot_general(w[:, h:], z, dn, preferred_element_type=jnp.float32)
    sb = s + b_ref[...][:, None]
    pad = jnp.zeros((2, s.shape[1]), jnp.float32)
    out_ref[...] = jnp.concatenate([sb, d, pad], axis=0)


def _make_sc_gather(n_nodes, n_edges):
    info = plsc.get_sparse_core_info()
    nc, ns = info.num_cores, info.num_subcores
    nw = nc * ns
    n_blocks = n_edges // _BLK
    base_blocks = n_blocks // nw
    n_extra = n_blocks - base_blocks * nw
    e_per_w = base_blocks * _BLK
    mesh = plsc.VectorSubcoreMesh(core_axis_name="c", subcore_axis_name="s")

    @functools.partial(
        pl.kernel,
        out_type=jax.ShapeDtypeStruct((4, n_edges), jnp.float32),
        mesh=mesh,
        scratch_types=[
            pltpu.VMEM((6 * n_nodes,), jnp.float32),
            pltpu.VMEM((base_blocks, 2, _BLK), jnp.int32),
            pltpu.VMEM((4, e_per_w), jnp.float32),
            pltpu.VMEM((1, 2, _BLK), jnp.int32),
            pltpu.VMEM((4, _BLK), jnp.float32),
            pltpu.SemaphoreType.DMA,
            pltpu.SemaphoreType.DMA,
        ],
        compiler_params=pltpu.CompilerParams(needs_layout_passes=False),
    )
    def sc_gather(tab_hbm, ei_hbm, out_hbm,
                  tab_v, ei_v, out_v, eit_v, outt_v, sem0, sem1):
        wid = lax.axis_index("s") * nc + lax.axis_index("c")
        bbase = wid * base_blocks
        cp_tab = pltpu.async_copy(tab_hbm.at[pl.ds(0, 6 * n_nodes)], tab_v,
                                  sem0)
        cp_idx = pltpu.async_copy(ei_hbm.at[pl.ds(bbase, base_blocks)], ei_v,
                                  sem1)
        cp_tab.wait()
        cp_idx.wait()

        def compute16(ev0, ev1, o_ref, col):
            s0 = plsc.load_gather(tab_v, [ev0])
            s1 = plsc.load_gather(tab_v, [ev0 + n_nodes])
            s2 = plsc.load_gather(tab_v, [ev0 + 2 * n_nodes])
            d0 = plsc.load_gather(tab_v, [ev1 + 3 * n_nodes])
            d1 = plsc.load_gather(tab_v, [ev1 + 4 * n_nodes])
            d2 = plsc.load_gather(tab_v, [ev1 + 5 * n_nodes])
            l0, l1, l2 = s0 + d0, s1 + d1, s2 + d2
            m = jnp.maximum(jnp.maximum(l0, l1), l2)
            ssum = (jnp.exp(l0 - m) + jnp.exp(l1 - m) + jnp.exp(l2 - m))
            bits = lax.bitcast_convert_type(ssum, jnp.int32)
            ex = (bits >> 23) - 127
            mf = lax.bitcast_convert_type(
                (bits & 0x007FFFFF) | 0x3F800000, jnp.float32)
            big = mf > _SQRT2
            mf = jnp.where(big, mf * 0.5, mf)
            exf = (ex + big.astype(jnp.int32)).astype(jnp.float32)
            t = (mf - 1.0) / (mf + 1.0)
            t2 = t * t
            p = ((t2 * (1.0 / 7.0) + (1.0 / 5.0)) * t2 + (1.0 / 3.0)) * t2 + 1.0
            lse = m + exf * _LN2 + 2.0 * t * p
            sl = pl.ds(col, _LANES)
            o_ref[0, sl] = l0 - lse
            o_ref[1, sl] = l1 - lse
            o_ref[2, sl] = l2 - lse

        gpb = _BLK // _LANES

        @plsc.parallel_loop(0, base_blocks * gpb, unroll=8)
        def _main(i):
            b = i // gpb
            g = i % gpb
            ev0 = ei_v[b, 0, pl.ds(g * _LANES, _LANES)]
            ev1 = ei_v[b, 1, pl.ds(g * _LANES, _LANES)]
            compute16(ev0, ev1, out_v, i * _LANES)
        cp_out = pltpu.async_copy(
            out_v, out_hbm.at[:, pl.ds(bbase * _BLK, e_per_w)], sem1)

        @pl.when(wid >= nw - n_extra)
        def _tail():
            tb = base_blocks * nw + (wid - (nw - n_extra))
            pltpu.async_copy(ei_hbm.at[pl.ds(tb, 1)], eit_v, sem0).wait()

            @plsc.parallel_loop(0, gpb, unroll=8)
            def _tail_loop(g):
                ev0 = eit_v[0, 0, pl.ds(g * _LANES, _LANES)]
                ev1 = eit_v[0, 1, pl.ds(g * _LANES, _LANES)]
                compute16(ev0, ev1, outt_v, g * _LANES)
            pltpu.async_copy(
                outt_v, out_hbm.at[:, pl.ds(tb * _BLK, _BLK)], sem0).wait()

        cp_out.wait()

    return sc_gather


def kernel(z, edge_index, W, b):
    n_nodes, hidden = z.shape
    n_edges = edge_index.shape[1]
    table = pl.pallas_call(
        _proj_body,
        out_shape=jax.ShapeDtypeStruct((8, n_nodes), jnp.float32),
    )(W, z, b)

    ei3 = (edge_index.astype(jnp.int32)
           .reshape(2, n_edges // _BLK, _BLK).transpose(1, 0, 2))
    planes = _make_sc_gather(n_nodes, n_edges)(table.reshape(-1), ei3)
    return planes[:3].T


# speedup vs baseline: 44.2397x; 1.2039x over previous
import functools

import jax
import jax.numpy as jnp
from jax import lax
from jax.experimental import pallas as pl
from jax.experimental.pallas import tpu as pltpu
from jax.experimental.pallas import tpu_sc as plsc

_LANES = 16
_BLK = 128
_LNC = (-1.8895877110388932, 3.385218192432659, -2.331779405969581,
        1.1162195608328185, -0.33073590701041583, 0.05458546334426203,
        -0.003832756714011353)


def _proj_body(w_ref, z_ref, b_ref, out_ref):
    w = w_ref[...]
    z = z_ref[...]
    h = w.shape[1] // 2
    dn = (((1,), (1,)), ((), ()))
    s = lax.dot_general(w[:, :h], z, dn, preferred_element_type=jnp.float32)
    d = lax.dot_general(w[:, h:], z, dn, preferred_element_type=jnp.float32)
    sb = s + b_ref[...][:, None]
    pad = jnp.zeros((2, s.shape[1]), jnp.float32)
    out_ref[...] = jnp.concatenate([sb, d, pad], axis=0)


def _make_sc_gather(n_nodes, n_edges):
    info = plsc.get_sparse_core_info()
    nc, ns = info.num_cores, info.num_subcores
    nw = nc * ns
    n_blocks = n_edges // _BLK
    base_blocks = n_blocks // nw
    n_extra = n_blocks - base_blocks * nw
    e_per_w = base_blocks * _BLK
    mesh = plsc.VectorSubcoreMesh(core_axis_name="c", subcore_axis_name="s")

    @functools.partial(
        pl.kernel,
        out_type=jax.ShapeDtypeStruct((3, n_edges), jnp.float32),
        mesh=mesh,
        scratch_types=[
            pltpu.VMEM((6 * n_nodes,), jnp.float32),
            pltpu.VMEM((base_blocks, 2, _BLK), jnp.int32),
            pltpu.VMEM((3, e_per_w), jnp.float32),
            pltpu.VMEM((1, 2, _BLK), jnp.int32),
            pltpu.VMEM((3, _BLK), jnp.float32),
            pltpu.SemaphoreType.DMA,
            pltpu.SemaphoreType.DMA,
        ],
        compiler_params=pltpu.CompilerParams(needs_layout_passes=False),
    )
    def sc_gather(tab_hbm, ei_hbm, out_hbm,
                  tab_v, ei_v, out_v, eit_v, outt_v, sem0, sem1):
        wid = lax.axis_index("s") * nc + lax.axis_index("c")
        bbase = wid * base_blocks
        cp_tab = pltpu.async_copy(tab_hbm.at[pl.ds(0, 6 * n_nodes)], tab_v,
                                  sem0)
        cp_idx = pltpu.async_copy(ei_hbm.at[pl.ds(bbase, base_blocks)], ei_v,
                                  sem1)
        cp_tab.wait()
        cp_idx.wait()

        def compute16(ev0, ev1, o_ref, col):
            s0 = plsc.load_gather(tab_v, [ev0])
            s1 = plsc.load_gather(tab_v, [ev0 + n_nodes])
            s2 = plsc.load_gather(tab_v, [ev0 + 2 * n_nodes])
            d0 = plsc.load_gather(tab_v, [ev1 + 3 * n_nodes])
            d1 = plsc.load_gather(tab_v, [ev1 + 4 * n_nodes])
            d2 = plsc.load_gather(tab_v, [ev1 + 5 * n_nodes])
            l0, l1, l2 = s0 + d0, s1 + d1, s2 + d2
            m = jnp.maximum(jnp.maximum(l0, l1), l2)
            ssum = (jnp.exp(l0 - m) + jnp.exp(l1 - m) + jnp.exp(l2 - m))
            p = _LNC[6]
            for c in (_LNC[5], _LNC[4], _LNC[3], _LNC[2], _LNC[1], _LNC[0]):
                p = p * ssum + c
            lse = m + p
            sl = pl.ds(col, _LANES)
            o_ref[0, sl] = l0 - lse
            o_ref[1, sl] = l1 - lse
            o_ref[2, sl] = l2 - lse

        gpb = _BLK // _LANES

        @plsc.parallel_loop(0, base_blocks * gpb, unroll=8)
        def _main(i):
            b = i // gpb
            g = i % gpb
            ev0 = ei_v[b, 0, pl.ds(g * _LANES, _LANES)]
            ev1 = ei_v[b, 1, pl.ds(g * _LANES, _LANES)]
            compute16(ev0, ev1, out_v, i * _LANES)
        cp_out = pltpu.async_copy(
            out_v, out_hbm.at[:, pl.ds(bbase * _BLK, e_per_w)], sem1)

        @pl.when(wid >= nw - n_extra)
        def _tail():
            tb = base_blocks * nw + (wid - (nw - n_extra))
            pltpu.async_copy(ei_hbm.at[pl.ds(tb, 1)], eit_v, sem0).wait()

            @plsc.parallel_loop(0, gpb, unroll=8)
            def _tail_loop(g):
                ev0 = eit_v[0, 0, pl.ds(g * _LANES, _LANES)]
                ev1 = eit_v[0, 1, pl.ds(g * _LANES, _LANES)]
                compute16(ev0, ev1, outt_v, g * _LANES)
            pltpu.async_copy(
                outt_v, out_hbm.at[:, pl.ds(tb * _BLK, _BLK)], sem0).wait()

        cp_out.wait()

    return sc_gather


def kernel(z, edge_index, W, b):
    n_nodes, hidden = z.shape
    n_edges = edge_index.shape[1]
    table = pl.pallas_call(
        _proj_body,
        out_shape=jax.ShapeDtypeStruct((8, n_nodes), jnp.float32),
    )(W, z, b)

    ei3 = (edge_index.astype(jnp.int32)
           .reshape(2, n_edges // _BLK, _BLK).transpose(1, 0, 2))
    planes = _make_sc_gather(n_nodes, n_edges)(table.reshape(-1), ei3)
    return planes.T
